# baseline (speedup 1.0000x reference)
import functools
import math

import jax
import jax.numpy as jnp
from jax import lax
from jax.experimental import pallas as pl
from jax.experimental.pallas import tpu as pltpu
from jax.experimental.pallas import tpu_sc as plsc

D_MODEL = 64
SCALE = math.sqrt(D_MODEL)

NUM_CORES = 2
NUM_SUBCORES = 16
NUM_WORKERS = NUM_CORES * NUM_SUBCORES

BLK = 128
SPS = 2
NBUF = 2


def _emb_body(xt_hbm, lut_hbm, out_hbm, idx_all,
              in0, in1, out0, out1, sg0, sg1, sw0, sw1,
              *, seq_len):
    wid = lax.axis_index("s") * NUM_CORES + lax.axis_index("c")
    num_chunks = seq_len // SPS
    rounds = num_chunks // NBUF

    ins = (in0, in1)
    outs = (out0, out1)
    sgs = (sg0, sg1)
    sws = (sw0, sw1)

    pltpu.sync_copy(xt_hbm.at[:, pl.ds(wid * BLK, BLK)], idx_all)

    iota = lax.iota(jnp.int32, 16)

    def fire_gather(i, b):
        for j in range(SPS):
            pltpu.async_copy(
                lut_hbm.at[idx_all.at[i * SPS + j]],
                ins[b].at[pl.ds(j * BLK, BLK)],
                sgs[b],
            )

    def wait_gather(i, b):
        for j in range(SPS):
            pltpu.make_async_copy(
                lut_hbm.at[idx_all.at[i * SPS + j]],
                ins[b].at[pl.ds(j * BLK, BLK)],
                sgs[b],
            ).wait()

    def fire_write(i, b):
        for j in range(SPS):
            pltpu.async_copy(
                outs[b].at[j], out_hbm.at[i * SPS + j, :, wid], sws[b]
            )

    def wait_write(i, b):
        for j in range(SPS):
            pltpu.make_async_copy(
                outs[b].at[j], out_hbm.at[i * SPS + j, :, wid], sws[b]
            ).wait()

    def transpose_scale(b):
        src = ins[b]
        dst = outs[b]

        def col(d, c):
            dt = lax.shift_right_logical(d, 3)
            dp = lax.bitwise_and(d, 7)
            cols = jnp.full((16,), d, jnp.int32)
            for j in range(SPS):
                for k in range(BLK // 16):
                    vals = plsc.load_gather(src, [j * BLK + k * 16 + iota, cols])
                    dst[j, dt, dp, pl.ds(k * 16, 16)] = vals * SCALE
            return c

        lax.fori_loop(0, D_MODEL, col, 0, unroll=2)

    for b in range(NBUF):
        fire_gather(b, b)
    for b in range(NBUF):
        wait_gather(b, b)
        transpose_scale(b)
        fire_write(b, b)
        fire_gather(b + NBUF, b)

    def round_body(g, c):
        for b in range(NBUF):
            i = g * NBUF + b
            wait_gather(i, b)
            wait_write(i - NBUF, b)
            transpose_scale(b)
            fire_write(i, b)
            fire_gather(i + NBUF, b)
        return c

    lax.fori_loop(1, rounds - 1, round_body, 0)

    for b in range(NBUF):
        i = num_chunks - NBUF + b
        wait_gather(i, b)
        wait_write(i - NBUF, b)
        transpose_scale(b)
        fire_write(i, b)
    for b in range(NBUF):
        wait_write(num_chunks - NBUF + b, b)


def kernel(x, lut):
    bsz, seq = x.shape
    assert bsz == NUM_WORKERS * BLK and seq % (SPS * NBUF) == 0
    xt = x.T

    mesh = plsc.VectorSubcoreMesh(core_axis_name="c", subcore_axis_name="s")
    run = pl.kernel(
        functools.partial(_emb_body, seq_len=seq),
        out_type=jax.ShapeDtypeStruct(
            (seq, D_MODEL // 8, NUM_WORKERS, 8, BLK), jnp.float32
        ),
        mesh=mesh,
        scratch_types=[
            pltpu.VMEM((seq, BLK), jnp.int32),
            pltpu.VMEM((SPS * BLK, D_MODEL), jnp.float32),
            pltpu.VMEM((SPS * BLK, D_MODEL), jnp.float32),
            pltpu.VMEM((SPS, D_MODEL // 8, 8, BLK), jnp.float32),
            pltpu.VMEM((SPS, D_MODEL // 8, 8, BLK), jnp.float32),
            pltpu.SemaphoreType.DMA,
            pltpu.SemaphoreType.DMA,
            pltpu.SemaphoreType.DMA,
            pltpu.SemaphoreType.DMA,
        ],
        compiler_params=pltpu.CompilerParams(
            use_tc_tiling_on_sc=False, needs_layout_passes=False
        ),
    )
    out5 = run(xt, lut)
    out = out5.transpose(2, 4, 0, 1, 3).reshape(bsz, seq, D_MODEL)
    return out


# speedup vs baseline: 1.7431x; 1.7431x over previous
import functools
import math

import jax
import jax.numpy as jnp
from jax import lax
from jax.experimental import pallas as pl
from jax.experimental.pallas import tpu as pltpu
from jax.experimental.pallas import tpu_sc as plsc

D_MODEL = 64
SCALE = math.sqrt(D_MODEL)

NUM_CORES = 2
NUM_SUBCORES = 16
NUM_WORKERS = NUM_CORES * NUM_SUBCORES

BLK = 128
PITCH = BLK + 1
SPS = 2
NBUF = 2


def _emb_body(xt_hbm, lut_hbm, out_hbm, idx_all,
              in0, in1, out0, out1, sg0, sg1, sw0, sw1,
              *, seq_len):
    wid = lax.axis_index("s") * NUM_CORES + lax.axis_index("c")
    num_chunks = seq_len // SPS
    rounds = num_chunks // NBUF

    ins = (in0, in1)
    outs = (out0, out1)
    sgs = (sg0, sg1)
    sws = (sw0, sw1)

    pltpu.sync_copy(xt_hbm.at[:, pl.ds(wid * BLK, BLK)], idx_all)

    iota = lax.iota(jnp.int32, 16)

    def fire_gather(i, b):
        for j in range(SPS):
            pltpu.async_copy(
                lut_hbm.at[idx_all.at[i * SPS + j]],
                ins[b].at[pl.ds(j * BLK, BLK)],
                sgs[b],
            )

    def wait_gather(i, b):
        for j in range(SPS):
            pltpu.make_async_copy(
                lut_hbm.at[idx_all.at[i * SPS + j]],
                ins[b].at[pl.ds(j * BLK, BLK)],
                sgs[b],
            ).wait()

    def fire_write(i, b):
        for j in range(SPS):
            pltpu.async_copy(
                outs[b].at[j, :, :, pl.ds(0, BLK)],
                out_hbm.at[i * SPS + j, :, wid],
                sws[b],
            )

    def wait_write(i, b):
        for j in range(SPS):
            pltpu.make_async_copy(
                outs[b].at[j, :, :, pl.ds(0, BLK)],
                out_hbm.at[i * SPS + j, :, wid],
                sws[b],
            ).wait()

    def transpose_scale(b):
        src = ins[b]
        dst = outs[b]

        def row(r, c):
            rv = jnp.full((16,), r, jnp.int32)
            for j in range(SPS):
                jv = jnp.full((16,), j, jnp.int32)
                for k in range(D_MODEL // 16):
                    d = k * 16 + iota
                    vals = src[j * BLK + r, pl.ds(k * 16, 16)]
                    plsc.store_scatter(
                        dst,
                        [jv, lax.shift_right_logical(d, 3), lax.bitwise_and(d, 7), rv],
                        vals * SCALE,
                    )
            return c

        lax.fori_loop(0, BLK, row, 0, unroll=4)

    for b in range(NBUF):
        fire_gather(b, b)
    for b in range(NBUF):
        wait_gather(b, b)
        transpose_scale(b)
        fire_write(b, b)
        fire_gather(b + NBUF, b)

    def round_body(g, c):
        for b in range(NBUF):
            i = g * NBUF + b
            wait_gather(i, b)
            wait_write(i - NBUF, b)
            transpose_scale(b)
            fire_write(i, b)
            fire_gather(i + NBUF, b)
        return c

    lax.fori_loop(1, rounds - 1, round_body, 0)

    for b in range(NBUF):
        i = num_chunks - NBUF + b
        wait_gather(i, b)
        wait_write(i - NBUF, b)
        transpose_scale(b)
        fire_write(i, b)
    for b in range(NBUF):
        wait_write(num_chunks - NBUF + b, b)


def kernel(x, lut):
    bsz, seq = x.shape
    assert bsz == NUM_WORKERS * BLK and seq % (SPS * NBUF) == 0
    xt = x.T

    mesh = plsc.VectorSubcoreMesh(core_axis_name="c", subcore_axis_name="s")
    run = pl.kernel(
        functools.partial(_emb_body, seq_len=seq),
        out_type=jax.ShapeDtypeStruct(
            (seq, D_MODEL // 8, NUM_WORKERS, 8, BLK), jnp.float32
        ),
        mesh=mesh,
        scratch_types=[
            pltpu.VMEM((seq, BLK), jnp.int32),
            pltpu.VMEM((SPS * BLK, D_MODEL), jnp.float32),
            pltpu.VMEM((SPS * BLK, D_MODEL), jnp.float32),
            pltpu.VMEM((SPS, D_MODEL // 8, 8, PITCH), jnp.float32),
            pltpu.VMEM((SPS, D_MODEL // 8, 8, PITCH), jnp.float32),
            pltpu.SemaphoreType.DMA,
            pltpu.SemaphoreType.DMA,
            pltpu.SemaphoreType.DMA,
            pltpu.SemaphoreType.DMA,
        ],
        compiler_params=pltpu.CompilerParams(
            use_tc_tiling_on_sc=False, needs_layout_passes=False
        ),
    )
    out5 = run(xt, lut)
    out = out5.transpose(2, 4, 0, 1, 3).reshape(bsz, seq, D_MODEL)
    return out


# speedup vs baseline: 1.8413x; 1.0563x over previous
import functools
import math

import jax
import jax.numpy as jnp
from jax import lax
from jax.experimental import pallas as pl
from jax.experimental.pallas import tpu as pltpu
from jax.experimental.pallas import tpu_sc as plsc

D_MODEL = 64
SCALE = math.sqrt(D_MODEL)

NUM_CORES = 2
NUM_SUBCORES = 16
NUM_WORKERS = NUM_CORES * NUM_SUBCORES

BLK = 128
PITCH = BLK + 1
HALF = D_MODEL // 2
SPS = 2
NBUF = 2


def _emb_body(xt_hbm, lut_hbm, out_hbm, idx_all,
              in0, in1, out0, out1, ix0, ix1,
              sg0, sg1, sw0, sw1, *, seq_len):
    wid = lax.axis_index("s") * NUM_CORES + lax.axis_index("c")
    num_chunks = seq_len // SPS
    rounds = num_chunks // NBUF

    lut32 = lut_hbm

    ins = (in0, in1)
    outs = (out0, out1)
    idxs = (ix0, ix1)
    sgs = (sg0, sg1)
    sws = (sw0, sw1)

    pltpu.sync_copy(xt_hbm.at[:, pl.ds(wid * BLK, BLK)], idx_all)

    iota = lax.iota(jnp.int32, 16)

    def prep(i, b):
        ix = idxs[b]
        for j in range(SPS):
            for t in range(BLK // 16):
                xv = idx_all[i * SPS + j, pl.ds(t * 16, 16)]
                ix[pl.ds(j * BLK + t * 16, 16)] = lax.shift_left(xv, 1)

    def fire_gather(i, b):
        for j in range(SPS):
            pltpu.async_copy(
                lut32.at[idxs[b].at[pl.ds(j * BLK, BLK)]],
                ins[b].at[pl.ds(j * BLK, BLK)],
                sgs[b],
            )

    def wait_gather(i, b):
        for j in range(SPS):
            pltpu.make_async_copy(
                lut32.at[idxs[b].at[pl.ds(j * BLK, BLK)]],
                ins[b].at[pl.ds(j * BLK, BLK)],
                sgs[b],
            ).wait()

    def fire_write(i, b):
        for j in range(SPS):
            pltpu.async_copy(
                outs[b].at[j, :, :, pl.ds(0, BLK)],
                out_hbm.at[i * SPS + j, :, wid],
                sws[b],
            )

    def wait_write(i, b):
        for j in range(SPS):
            pltpu.make_async_copy(
                outs[b].at[j, :, :, pl.ds(0, BLK)],
                out_hbm.at[i * SPS + j, :, wid],
                sws[b],
            ).wait()

    def transpose_scale(b):
        src = ins[b]
        dst = outs[b]

        def row(r, c):
            rv = jnp.full((16,), r, jnp.int32)
            for j in range(SPS):
                jv = jnp.full((16,), j, jnp.int32)
                for k in range(D_MODEL // 16):
                    d = k * 16 + iota
                    vals = src[j * BLK + r, pl.ds(k * 16, 16)]
                    plsc.store_scatter(
                        dst,
                        [jv, lax.shift_right_logical(d, 3), lax.bitwise_and(d, 7), rv],
                        vals * SCALE,
                    )
            return c

        lax.fori_loop(0, BLK, row, 0, unroll=4)

    for b in range(NBUF):
        prep(b, b)
        fire_gather(b, b)
    for b in range(NBUF):
        wait_gather(b, b)
        transpose_scale(b)
        fire_write(b, b)
        prep(b + NBUF, b)
        fire_gather(b + NBUF, b)

    def round_body(g, c):
        for b in range(NBUF):
            i = g * NBUF + b
            wait_gather(i, b)
            wait_write(i - NBUF, b)
            transpose_scale(b)
            fire_write(i, b)
            prep(i + NBUF, b)
            fire_gather(i + NBUF, b)
        return c

    lax.fori_loop(1, rounds - 1, round_body, 0)

    for b in range(NBUF):
        i = num_chunks - NBUF + b
        wait_gather(i, b)
        wait_write(i - NBUF, b)
        transpose_scale(b)
        fire_write(i, b)
    for b in range(NBUF):
        wait_write(num_chunks - NBUF + b, b)


def kernel(x, lut):
    bsz, seq = x.shape
    vocab, dm = lut.shape
    assert bsz == NUM_WORKERS * BLK and seq % (SPS * NBUF) == 0 and dm == D_MODEL
    xt = x.T
    lut2 = jnp.pad(lut, ((0, 0), (0, D_MODEL))).reshape(2 * vocab, D_MODEL)

    mesh = plsc.VectorSubcoreMesh(core_axis_name="c", subcore_axis_name="s")
    run = pl.kernel(
        functools.partial(_emb_body, seq_len=seq),
        out_type=jax.ShapeDtypeStruct(
            (seq, D_MODEL // 8, NUM_WORKERS, 8, BLK), jnp.float32
        ),
        mesh=mesh,
        scratch_types=[
            pltpu.VMEM((seq, BLK), jnp.int32),
            pltpu.VMEM((SPS * BLK, D_MODEL), jnp.float32),
            pltpu.VMEM((SPS * BLK, D_MODEL), jnp.float32),
            pltpu.VMEM((SPS, D_MODEL // 8, 8, PITCH), jnp.float32),
            pltpu.VMEM((SPS, D_MODEL // 8, 8, PITCH), jnp.float32),
            pltpu.VMEM((SPS * BLK,), jnp.int32),
            pltpu.VMEM((SPS * BLK,), jnp.int32),
            pltpu.SemaphoreType.DMA,
            pltpu.SemaphoreType.DMA,
            pltpu.SemaphoreType.DMA,
            pltpu.SemaphoreType.DMA,
        ],
        compiler_params=pltpu.CompilerParams(
            use_tc_tiling_on_sc=False, needs_layout_passes=False
        ),
    )
    out5 = run(xt, lut2)
    out = out5.transpose(2, 4, 0, 1, 3).reshape(bsz, seq, D_MODEL)
    return out


# speedup vs baseline: 2.8132x; 1.5279x over previous
import functools
import math

import jax
import jax.numpy as jnp
from jax import lax
from jax.experimental import pallas as pl
from jax.experimental.pallas import tpu as pltpu
from jax.experimental.pallas import tpu_sc as plsc

D_MODEL = 64
SCALE = math.sqrt(D_MODEL)

NUM_CORES = 2
NUM_SUBCORES = 16
NUM_WORKERS = NUM_CORES * NUM_SUBCORES

BLK = 128
PITCH = BLK + 1
HALF = D_MODEL // 2
SPS = 2
NBUF = 2


def _emb_body(xt_hbm, lut_hbm, out_hbm, idx_all,
              in0, in1, out0, out1, ix0, ix1,
              sg0, sg1, sw0, sw1, *, seq_len):
    wid = lax.axis_index("s") * NUM_CORES + lax.axis_index("c")
    num_chunks = seq_len // SPS
    rounds = num_chunks // NBUF

    lut32 = lut_hbm

    ins = (in0, in1)
    outs = (out0, out1)
    idxs = (ix0, ix1)
    sgs = (sg0, sg1)
    sws = (sw0, sw1)

    pltpu.sync_copy(xt_hbm.at[:, pl.ds(wid * BLK, BLK)], idx_all)

    iota = lax.iota(jnp.int32, 16)

    def prep(i, b):
        ix = idxs[b]
        for j in range(SPS):
            for t in range(BLK // 16):
                xv = idx_all[i * SPS + j, pl.ds(t * 16, 16)]
                ix[pl.ds(j * BLK + t * 16, 16)] = lax.shift_left(xv, 1)

    def fire_gather(i, b):
        for j in range(SPS):
            pltpu.async_copy(
                lut32.at[idxs[b].at[pl.ds(j * BLK, BLK)]],
                ins[b].at[pl.ds(j * BLK, BLK)],
                sgs[b],
            )

    def wait_gather(i, b):
        for j in range(SPS):
            pltpu.make_async_copy(
                lut32.at[idxs[b].at[pl.ds(j * BLK, BLK)]],
                ins[b].at[pl.ds(j * BLK, BLK)],
                sgs[b],
            ).wait()

    def fire_write(i, b):
        for j in range(SPS):
            pltpu.async_copy(
                outs[b].at[j, :, :, pl.ds(0, BLK)],
                out_hbm.at[i * SPS + j, :, wid],
                sws[b],
            )

    def wait_write(i, b):
        for j in range(SPS):
            pltpu.make_async_copy(
                outs[b].at[j, :, :, pl.ds(0, BLK)],
                out_hbm.at[i * SPS + j, :, wid],
                sws[b],
            ).wait()

    def transpose_scale(b):
        src = ins[b]
        dst = outs[b]

        @plsc.parallel_loop(0, BLK, step=1, unroll=4)
        def row(r):
            rv = jnp.full((16,), r, jnp.int32)
            for j in range(SPS):
                jv = jnp.full((16,), j, jnp.int32)
                for k in range(D_MODEL // 16):
                    d = k * 16 + iota
                    vals = src[j * BLK + r, pl.ds(k * 16, 16)]
                    plsc.store_scatter(
                        dst,
                        [jv, lax.shift_right_logical(d, 3), lax.bitwise_and(d, 7), rv],
                        vals * SCALE,
                    )

    for b in range(NBUF):
        prep(b, b)
        fire_gather(b, b)
    for b in range(NBUF):
        wait_gather(b, b)
        transpose_scale(b)
        fire_write(b, b)
        prep(b + NBUF, b)
        fire_gather(b + NBUF, b)

    def round_body(g, c):
        for b in range(NBUF):
            i = g * NBUF + b
            wait_gather(i, b)
            wait_write(i - NBUF, b)
            transpose_scale(b)
            fire_write(i, b)
            prep(i + NBUF, b)
            fire_gather(i + NBUF, b)
        return c

    lax.fori_loop(1, rounds - 1, round_body, 0)

    for b in range(NBUF):
        i = num_chunks - NBUF + b
        wait_gather(i, b)
        wait_write(i - NBUF, b)
        transpose_scale(b)
        fire_write(i, b)
    for b in range(NBUF):
        wait_write(num_chunks - NBUF + b, b)


def kernel(x, lut):
    bsz, seq = x.shape
    vocab, dm = lut.shape
    assert bsz == NUM_WORKERS * BLK and seq % (SPS * NBUF) == 0 and dm == D_MODEL
    xt = x.T
    lut2 = jnp.pad(lut, ((0, 0), (0, D_MODEL))).reshape(2 * vocab, D_MODEL)

    mesh = plsc.VectorSubcoreMesh(core_axis_name="c", subcore_axis_name="s")
    run = pl.kernel(
        functools.partial(_emb_body, seq_len=seq),
        out_type=jax.ShapeDtypeStruct(
            (seq, D_MODEL // 8, NUM_WORKERS, 8, BLK), jnp.float32
        ),
        mesh=mesh,
        scratch_types=[
            pltpu.VMEM((seq, BLK), jnp.int32),
            pltpu.VMEM((SPS * BLK, D_MODEL), jnp.float32),
            pltpu.VMEM((SPS * BLK, D_MODEL), jnp.float32),
            pltpu.VMEM((SPS, D_MODEL // 8, 8, PITCH), jnp.float32),
            pltpu.VMEM((SPS, D_MODEL // 8, 8, PITCH), jnp.float32),
            pltpu.VMEM((SPS * BLK,), jnp.int32),
            pltpu.VMEM((SPS * BLK,), jnp.int32),
            pltpu.SemaphoreType.DMA,
            pltpu.SemaphoreType.DMA,
            pltpu.SemaphoreType.DMA,
            pltpu.SemaphoreType.DMA,
        ],
        compiler_params=pltpu.CompilerParams(
            use_tc_tiling_on_sc=False, needs_layout_passes=False
        ),
    )
    out5 = run(xt, lut2)
    out = out5.transpose(2, 4, 0, 1, 3).reshape(bsz, seq, D_MODEL)
    return out
